# baseline (speedup 1.0000x reference)
import functools

import jax
import jax.numpy as jnp
from jax import lax
from jax.experimental import pallas as pl
from jax.experimental.pallas import tpu as pltpu
from jax.experimental.pallas import tpu_sc as plsc

N = 10000
E = 320000
D = 128
HD0, F0 = 8, 8
F1 = 40
NEG = 0.2

NC, NS, L = 2, 16, 16
NW = NC * NS
EPW = E // NW
CH = 80
NCHUNK = EPW // CH

RB = 400
GRID = N // RB

HEW = 80
P0W = 80
P1W = 48
DUMP = 640

_SC_PARAMS = pltpu.CompilerParams(use_tc_tiling_on_sc=False)


def _k1_body(x_ref, w_ref, al_ref, ar_ref, he_ref, er_ref, m_ref):
    h = jnp.dot(x_ref[...], w_ref[...], preferred_element_type=jnp.float32)
    el = jnp.dot(h, al_ref[...], preferred_element_type=jnp.float32)
    er = jnp.dot(h, ar_ref[...], preferred_element_type=jnp.float32)
    he_ref[...] = jnp.concatenate([h, el, el], axis=1)
    er_ref[...] = jnp.concatenate([er, er], axis=1)
    cat = jnp.concatenate([el, er], axis=1)
    m_ref[...] = jnp.max(cat, axis=0, keepdims=True)[None]


def _k1_call(x, W0p, alp, arp):
    return pl.pallas_call(
        _k1_body,
        grid=(GRID,),
        in_specs=[
            pl.BlockSpec((RB, D), lambda i: (i, 0)),
            pl.BlockSpec((D, HD0 * F0), lambda i: (0, 0)),
            pl.BlockSpec((HD0 * F0, HD0), lambda i: (0, 0)),
            pl.BlockSpec((HD0 * F0, HD0), lambda i: (0, 0)),
        ],
        out_specs=[
            pl.BlockSpec((RB, HEW), lambda i: (i, 0)),
            pl.BlockSpec((RB, 16), lambda i: (i, 0)),
            pl.BlockSpec((1, 1, 16), lambda i: (i, 0, 0)),
        ],
        out_shape=[
            jax.ShapeDtypeStruct((N, HEW), jnp.float32),
            jax.ShapeDtypeStruct((N, 16), jnp.float32),
            jax.ShapeDtypeStruct((GRID, 1, 16), jnp.float32),
        ],
    )(x, W0p, alp, arp)


def _k3_body(a_ref, b0_ref, w1_ref, al1_ref, ar1_ref, h1_ref, er1_ref, m_ref):
    a = a_ref[...]
    t = a[0] + a[1]
    s = t[:, 0:8]
    num = t[:, 16:P0W]
    srep = jnp.broadcast_to(s[:, None, :], (RB, 8, 8)).reshape(RB, 64)
    rst = jnp.maximum(num / (srep + 1e-9) + b0_ref[...][None, :], 0.0)
    h1 = jnp.dot(rst, w1_ref[...], preferred_element_type=jnp.float32)
    el1 = jnp.sum(h1 * al1_ref[...][0][None, :], axis=1, keepdims=True)
    er1 = jnp.sum(h1 * ar1_ref[...][0][None, :], axis=1, keepdims=True)
    h1_ref[...] = jnp.concatenate(
        [h1, el1, jnp.zeros((RB, 7), jnp.float32)], axis=1)
    er1_ref[...] = jnp.concatenate(
        [er1, jnp.zeros((RB, 15), jnp.float32)], axis=1)
    m = jnp.concatenate(
        [jnp.max(el1).reshape(1), jnp.max(er1).reshape(1),
         jnp.zeros((14,), jnp.float32)])
    m_ref[...] = m.reshape(1, 1, 16)


def _k3_call(acc0, b0, W1, al1, ar1):
    return pl.pallas_call(
        _k3_body,
        grid=(GRID,),
        in_specs=[
            pl.BlockSpec((2, RB, P0W), lambda i: (0, i, 0)),
            pl.BlockSpec((HD0 * F0,), lambda i: (0,)),
            pl.BlockSpec((HD0 * F0, F1), lambda i: (0, 0)),
            pl.BlockSpec((1, F1), lambda i: (0, 0)),
            pl.BlockSpec((1, F1), lambda i: (0, 0)),
        ],
        out_specs=[
            pl.BlockSpec((RB, P1W), lambda i: (i, 0)),
            pl.BlockSpec((RB, 16), lambda i: (i, 0)),
            pl.BlockSpec((1, 1, 16), lambda i: (i, 0, 0)),
        ],
        out_shape=[
            jax.ShapeDtypeStruct((N, P1W), jnp.float32),
            jax.ShapeDtypeStruct((N, 16), jnp.float32),
            jax.ShapeDtypeStruct((GRID, 1, 16), jnp.float32),
        ],
    )(acc0, b0, W1, al1, ar1)


def _k5_body(a_ref, b1_ref, o_ref):
    a = a_ref[...]
    t = a[0] + a[1]
    num = t[:, 0:F1]
    s = t[:, F1:F1 + 1]
    o_ref[...] = num / (s + 1e-9) + b1_ref[...][None, :]


def _k5_call(acc1, b1):
    return pl.pallas_call(
        _k5_body,
        grid=(GRID,),
        in_specs=[
            pl.BlockSpec((2, RB, P1W), lambda i: (0, i, 0)),
            pl.BlockSpec((F1,), lambda i: (0,)),
        ],
        out_specs=pl.BlockSpec((RB, F1), lambda i: (i, 0)),
        out_shape=jax.ShapeDtypeStruct((N, F1), jnp.float32),
    )(acc1, b1)


def _sc0_body(he_hbm, er_hbm, src_hbm, dst_hbm, cvec_hbm, z_hbm, out_hbm,
              acc, srci, dsti, he0, he1, er0, er1, p0, p1, cvec_v,
              semg0, semg1, sems0, sems1):
    cid = lax.axis_index("c")
    sid = lax.axis_index("s")
    w = cid * NS + sid
    he_b, er_b, p_b = (he0, he1), (er0, er1), (p0, p1)
    semg, sems = (semg0, semg1), (sems0, sems1)

    zbase = jnp.minimum(sid * DUMP, N - DUMP)
    pltpu.sync_copy(z_hbm.at[pl.ds(zbase, DUMP)], acc.at[pl.ds(zbase, DUMP)])
    pltpu.sync_copy(cvec_hbm, cvec_v)
    pltpu.sync_copy(src_hbm.at[w], srci)
    pltpu.sync_copy(dst_hbm.at[w], dsti)
    plsc.subcore_barrier()

    def issue_g(b, k):
        pltpu.async_copy(he_hbm.at[srci.at[k]], he_b[b], semg[b])
        pltpu.async_copy(er_hbm.at[dsti.at[k]], er_b[b], semg[b])

    def wait_g(b, k):
        pltpu.make_async_copy(he_hbm.at[srci.at[k]], he_b[b], semg[b]).wait()
        pltpu.make_async_copy(er_hbm.at[dsti.at[k]], er_b[b], semg[b]).wait()

    def issue_s(b, k):
        pltpu.async_copy(p_b[b], acc.at[dsti.at[k]], sems[b], add=True)

    def wait_s(b, k):
        pltpu.make_async_copy(
            p_b[b], acc.at[dsti.at[k]], sems[b]).wait()

    def compute(b, k):
        cv = cvec_v[...]
        he_v, er_v, p_v = he_b[b], er_b[b], p_b[b]
        for i in range(CH):
            el = he_v[i, pl.ds(64, 16)]
            er = er_v[i, pl.ds(0, 16)]
            e = el + er
            e = jnp.maximum(e, NEG * e)
            ex = jnp.exp(e - cv)
            p_v[i, pl.ds(0, 16)] = ex
            for q in range(4):
                p_v[i, pl.ds(16 + 16 * q, 16)] = (
                    he_v[i, pl.ds(16 * q, 16)] * ex)

    issue_g(0, 0)

    def pair(g, _):
        for b in range(2):
            k = 2 * g + b
            wait_g(b, k)
            issue_g(1 - b, k + 1)
            pl.when(g > 0)(lambda: wait_s(b, k))
            compute(b, k)
            issue_s(b, k)
        return ()

    lax.fori_loop(0, (NCHUNK - 1) // 2, pair, (), unroll=False)
    kt = NCHUNK - 1
    wait_g(0, kt)
    wait_s(0, kt)
    compute(0, kt)
    issue_s(0, kt)
    wait_s(0, kt)
    wait_s(1, kt)
    plsc.subcore_barrier()
    dbase = jnp.minimum(sid * DUMP, N - DUMP)
    pltpu.sync_copy(acc.at[pl.ds(dbase, DUMP)],
                    out_hbm.at[cid, pl.ds(dbase, DUMP)])


def _sc0_call(he, er, src3, dst3, cvec, z0):
    mesh = plsc.VectorSubcoreMesh(core_axis_name="c", subcore_axis_name="s")
    f = pl.kernel(
        _sc0_body,
        out_type=jax.ShapeDtypeStruct((NC, N, P0W), jnp.float32),
        mesh=mesh,
        compiler_params=_SC_PARAMS,
        scratch_types=[
            pltpu.VMEM_SHARED((N, P0W), jnp.float32),
            pltpu.VMEM((NCHUNK, CH), jnp.int32),
            pltpu.VMEM((NCHUNK, CH), jnp.int32),
            pltpu.VMEM((CH, HEW), jnp.float32),
            pltpu.VMEM((CH, HEW), jnp.float32),
            pltpu.VMEM((CH, 16), jnp.float32),
            pltpu.VMEM((CH, 16), jnp.float32),
            pltpu.VMEM((CH, P0W), jnp.float32),
            pltpu.VMEM((CH, P0W), jnp.float32),
            pltpu.VMEM((16,), jnp.float32),
            pltpu.SemaphoreType.DMA,
            pltpu.SemaphoreType.DMA,
            pltpu.SemaphoreType.DMA,
            pltpu.SemaphoreType.DMA,
        ],
    )
    return f(he, er, src3, dst3, cvec, z0)


def _sc1_body(h1_hbm, er1_hbm, src_hbm, dst_hbm, cvec_hbm, z_hbm, out_hbm,
              acc, srci, dsti, h10, h11, er0, er1, p0, p1, cvec_v,
              semg0, semg1, sems0, sems1):
    cid = lax.axis_index("c")
    sid = lax.axis_index("s")
    w = cid * NS + sid
    h1_b, er_b, p_b = (h10, h11), (er0, er1), (p0, p1)
    semg, sems = (semg0, semg1), (sems0, sems1)

    zbase = jnp.minimum(sid * DUMP, N - DUMP)
    pltpu.sync_copy(z_hbm.at[pl.ds(zbase, DUMP)], acc.at[pl.ds(zbase, DUMP)])
    pltpu.sync_copy(cvec_hbm, cvec_v)
    pltpu.sync_copy(src_hbm.at[w], srci)
    pltpu.sync_copy(dst_hbm.at[w], dsti)
    plsc.subcore_barrier()

    iot = lax.iota(jnp.int32, 16)
    zv = jnp.zeros((16,), jnp.float32)

    def issue_g(b, k):
        pltpu.async_copy(h1_hbm.at[srci.at[k]], h1_b[b], semg[b])
        pltpu.async_copy(er1_hbm.at[dsti.at[k]], er_b[b], semg[b])

    def wait_g(b, k):
        pltpu.make_async_copy(h1_hbm.at[srci.at[k]], h1_b[b], semg[b]).wait()
        pltpu.make_async_copy(er1_hbm.at[dsti.at[k]], er_b[b], semg[b]).wait()

    def issue_s(b, k):
        pltpu.async_copy(p_b[b], acc.at[dsti.at[k]], sems[b], add=True)

    def wait_s(b, k):
        pltpu.make_async_copy(
            p_b[b], acc.at[dsti.at[k]], sems[b]).wait()

    def compute(b, k):
        cv = cvec_v[...]
        cs = cv[0]
        h1_v, er1_v, p_v = h1_b[b], er_b[b], p_b[b]
        for i in range(CH):
            h3 = h1_v[i, pl.ds(32, 16)]
            erv = er1_v[i, pl.ds(0, 16)]
            e0 = h3[8] + erv[0]
            e0 = jnp.maximum(e0, NEG * e0) - cs
            exv = jnp.exp(jnp.full((16,), e0, jnp.float32))
            p_v[i, pl.ds(0, 16)] = h1_v[i, pl.ds(0, 16)] * exv
            p_v[i, pl.ds(16, 16)] = h1_v[i, pl.ds(16, 16)] * exv
            p_v[i, pl.ds(32, 16)] = jnp.where(
                iot < 8, h3 * exv, jnp.where(iot == 8, exv, zv))

    issue_g(0, 0)

    def pair(g, _):
        for b in range(2):
            k = 2 * g + b
            wait_g(b, k)
            issue_g(1 - b, k + 1)
            pl.when(g > 0)(lambda: wait_s(b, k))
            compute(b, k)
            issue_s(b, k)
        return ()

    lax.fori_loop(0, (NCHUNK - 1) // 2, pair, (), unroll=False)
    kt = NCHUNK - 1
    wait_g(0, kt)
    wait_s(0, kt)
    compute(0, kt)
    issue_s(0, kt)
    wait_s(0, kt)
    wait_s(1, kt)
    plsc.subcore_barrier()
    dbase = jnp.minimum(sid * DUMP, N - DUMP)
    pltpu.sync_copy(acc.at[pl.ds(dbase, DUMP)],
                    out_hbm.at[cid, pl.ds(dbase, DUMP)])


def _sc1_call(h1, er1, src3, dst3, cvec, z1):
    mesh = plsc.VectorSubcoreMesh(core_axis_name="c", subcore_axis_name="s")
    f = pl.kernel(
        _sc1_body,
        out_type=jax.ShapeDtypeStruct((NC, N, P1W), jnp.float32),
        mesh=mesh,
        compiler_params=_SC_PARAMS,
        scratch_types=[
            pltpu.VMEM_SHARED((N, P1W), jnp.float32),
            pltpu.VMEM((NCHUNK, CH), jnp.int32),
            pltpu.VMEM((NCHUNK, CH), jnp.int32),
            pltpu.VMEM((CH, P1W), jnp.float32),
            pltpu.VMEM((CH, P1W), jnp.float32),
            pltpu.VMEM((CH, 16), jnp.float32),
            pltpu.VMEM((CH, 16), jnp.float32),
            pltpu.VMEM((CH, P1W), jnp.float32),
            pltpu.VMEM((CH, P1W), jnp.float32),
            pltpu.VMEM((16,), jnp.float32),
            pltpu.SemaphoreType.DMA,
            pltpu.SemaphoreType.DMA,
            pltpu.SemaphoreType.DMA,
            pltpu.SemaphoreType.DMA,
        ],
    )
    return f(h1, er1, src3, dst3, cvec, z1)


def kernel(x, edge_index, W0, attn_l0, attn_r0, b0, W1, attn_l1, attn_r1,
           b1):
    src3 = edge_index[0].reshape(NW, NCHUNK, CH)
    dst3 = edge_index[1].reshape(NW, NCHUNK, CH)

    j = jnp.arange(HD0 * F0)
    perm = (j % HD0) * F0 + j // HD0
    W0p = W0[:, perm]
    hd_of = j % HD0
    alp = jnp.zeros((HD0 * F0, HD0), jnp.float32).at[j, hd_of].set(
        attn_l0[hd_of, j // HD0])
    arp = jnp.zeros((HD0 * F0, HD0), jnp.float32).at[j, hd_of].set(
        attn_r0[hd_of, j // HD0])
    b0p = b0[perm]
    W1p = W1[perm, :]

    he, er, m0 = _k1_call(x, W0p, alp, arp)
    c0 = jnp.max(m0[:, 0, 0:8], axis=0) + jnp.max(m0[:, 0, 8:16], axis=0)
    c0 = jnp.maximum(c0, NEG * c0)
    cvec0 = jnp.concatenate([c0, c0])

    z0 = jnp.zeros((N, P0W), jnp.float32)
    acc0 = _sc0_call(he, er, src3, dst3, cvec0, z0)

    h1, er1, m1 = _k3_call(acc0, b0p, W1p, attn_l1, attn_r1)
    c1 = jnp.max(m1[:, 0, 0]) + jnp.max(m1[:, 0, 1])
    c1 = jnp.maximum(c1, NEG * c1)
    cvec1 = jnp.full((16,), c1, jnp.float32)

    z1 = jnp.zeros((N, P1W), jnp.float32)
    acc1 = _sc1_call(h1, er1, src3, dst3, cvec1, z1)

    return _k5_call(acc1, b1)


# speedup vs baseline: 1.1065x; 1.1065x over previous
import functools

import jax
import jax.numpy as jnp
from jax import lax
from jax.experimental import pallas as pl
from jax.experimental.pallas import tpu as pltpu
from jax.experimental.pallas import tpu_sc as plsc

N = 10000
E = 320000
D = 128
HD0, F0 = 8, 8
F1 = 40
NEG = 0.2

NC, NS, L = 2, 16, 16
NW = NC * NS
EPW = E // NW
CH = 125
NCHUNK = EPW // CH

RB = 400
GRID = N // RB

HEW = 80
P0W = 80
P1W = 48
DUMP = 640

_SC_PARAMS = pltpu.CompilerParams(use_tc_tiling_on_sc=False)


def _k1_body(x_ref, w_ref, al_ref, ar_ref, he_ref, er_ref, m_ref):
    h = jnp.dot(x_ref[...], w_ref[...], preferred_element_type=jnp.float32)
    el = jnp.dot(h, al_ref[...], preferred_element_type=jnp.float32)
    er = jnp.dot(h, ar_ref[...], preferred_element_type=jnp.float32)
    he_ref[...] = jnp.concatenate([h, el, el], axis=1)
    er_ref[...] = jnp.concatenate([er, er], axis=1)
    cat = jnp.concatenate([el, er], axis=1)
    m_ref[...] = jnp.max(cat, axis=0, keepdims=True)[None]


def _k1_call(x, W0p, alp, arp):
    return pl.pallas_call(
        _k1_body,
        grid=(GRID,),
        in_specs=[
            pl.BlockSpec((RB, D), lambda i: (i, 0)),
            pl.BlockSpec((D, HD0 * F0), lambda i: (0, 0)),
            pl.BlockSpec((HD0 * F0, HD0), lambda i: (0, 0)),
            pl.BlockSpec((HD0 * F0, HD0), lambda i: (0, 0)),
        ],
        out_specs=[
            pl.BlockSpec((RB, HEW), lambda i: (i, 0)),
            pl.BlockSpec((RB, 16), lambda i: (i, 0)),
            pl.BlockSpec((1, 1, 16), lambda i: (i, 0, 0)),
        ],
        out_shape=[
            jax.ShapeDtypeStruct((N, HEW), jnp.float32),
            jax.ShapeDtypeStruct((N, 16), jnp.float32),
            jax.ShapeDtypeStruct((GRID, 1, 16), jnp.float32),
        ],
    )(x, W0p, alp, arp)


def _k3_body(a_ref, b0_ref, w1_ref, al1_ref, ar1_ref, h1_ref, er1_ref, m_ref):
    a = a_ref[...]
    t = a[0] + a[1]
    s = t[:, 0:8]
    num = t[:, 16:P0W]
    srep = jnp.broadcast_to(s[:, None, :], (RB, 8, 8)).reshape(RB, 64)
    rst = jnp.maximum(num / (srep + 1e-9) + b0_ref[...][None, :], 0.0)
    h1 = jnp.dot(rst, w1_ref[...], preferred_element_type=jnp.float32)
    el1 = jnp.sum(h1 * al1_ref[...][0][None, :], axis=1, keepdims=True)
    er1 = jnp.sum(h1 * ar1_ref[...][0][None, :], axis=1, keepdims=True)
    h1_ref[...] = jnp.concatenate(
        [h1, el1, jnp.zeros((RB, 7), jnp.float32)], axis=1)
    er1_ref[...] = jnp.concatenate(
        [er1, jnp.zeros((RB, 15), jnp.float32)], axis=1)
    m = jnp.concatenate(
        [jnp.max(el1).reshape(1), jnp.max(er1).reshape(1),
         jnp.zeros((14,), jnp.float32)])
    m_ref[...] = m.reshape(1, 1, 16)


def _k3_call(acc0, b0, W1, al1, ar1):
    return pl.pallas_call(
        _k3_body,
        grid=(GRID,),
        in_specs=[
            pl.BlockSpec((2, RB, P0W), lambda i: (0, i, 0)),
            pl.BlockSpec((HD0 * F0,), lambda i: (0,)),
            pl.BlockSpec((HD0 * F0, F1), lambda i: (0, 0)),
            pl.BlockSpec((1, F1), lambda i: (0, 0)),
            pl.BlockSpec((1, F1), lambda i: (0, 0)),
        ],
        out_specs=[
            pl.BlockSpec((RB, P1W), lambda i: (i, 0)),
            pl.BlockSpec((RB, 16), lambda i: (i, 0)),
            pl.BlockSpec((1, 1, 16), lambda i: (i, 0, 0)),
        ],
        out_shape=[
            jax.ShapeDtypeStruct((N, P1W), jnp.float32),
            jax.ShapeDtypeStruct((N, 16), jnp.float32),
            jax.ShapeDtypeStruct((GRID, 1, 16), jnp.float32),
        ],
    )(acc0, b0, W1, al1, ar1)


def _k5_body(a_ref, b1_ref, o_ref):
    a = a_ref[...]
    t = a[0] + a[1]
    num = t[:, 0:F1]
    s = t[:, F1:F1 + 1]
    o_ref[...] = num / (s + 1e-9) + b1_ref[...][None, :]


def _k5_call(acc1, b1):
    return pl.pallas_call(
        _k5_body,
        grid=(GRID,),
        in_specs=[
            pl.BlockSpec((2, RB, P1W), lambda i: (0, i, 0)),
            pl.BlockSpec((F1,), lambda i: (0,)),
        ],
        out_specs=pl.BlockSpec((RB, F1), lambda i: (i, 0)),
        out_shape=jax.ShapeDtypeStruct((N, F1), jnp.float32),
    )(acc1, b1)


def _sc0_body(he_hbm, er_hbm, src_hbm, dst_hbm, cvec_hbm, z_hbm, out_hbm,
              acc, srci, dsti, he0, he1, er0, er1, p0, p1, cvec_v,
              semg0, semg1, sems0, sems1):
    cid = lax.axis_index("c")
    sid = lax.axis_index("s")
    w = cid * NS + sid
    he_b, er_b, p_b = (he0, he1), (er0, er1), (p0, p1)
    semg, sems = (semg0, semg1), (sems0, sems1)

    zbase = jnp.minimum(sid * DUMP, N - DUMP)
    pltpu.sync_copy(z_hbm.at[pl.ds(zbase, DUMP)], acc.at[pl.ds(zbase, DUMP)])
    pltpu.sync_copy(cvec_hbm, cvec_v)
    pltpu.sync_copy(src_hbm.at[w], srci)
    pltpu.sync_copy(dst_hbm.at[w], dsti)
    plsc.subcore_barrier()

    def issue_g(b, k):
        pltpu.async_copy(he_hbm.at[srci.at[k]], he_b[b], semg[b])
        pltpu.async_copy(er_hbm.at[dsti.at[k]], er_b[b], semg[b])

    def wait_g(b, k):
        pltpu.make_async_copy(he_hbm.at[srci.at[k]], he_b[b], semg[b]).wait()
        pltpu.make_async_copy(er_hbm.at[dsti.at[k]], er_b[b], semg[b]).wait()

    def issue_s(b, k):
        pltpu.async_copy(p_b[b], acc.at[dsti.at[k]], sems[b], add=True)

    def wait_s(b, k):
        pltpu.make_async_copy(
            p_b[b], acc.at[dsti.at[k]], sems[b]).wait()

    def compute(b, k):
        cv = cvec_v[...]
        he_v, er_v, p_v = he_b[b], er_b[b], p_b[b]
        for i in range(CH):
            el = he_v[i, pl.ds(64, 16)]
            er = er_v[i, pl.ds(0, 16)]
            e = el + er
            e = jnp.maximum(e, NEG * e)
            ex = jnp.exp(e - cv)
            p_v[i, pl.ds(0, 16)] = ex
            for q in range(4):
                p_v[i, pl.ds(16 + 16 * q, 16)] = (
                    he_v[i, pl.ds(16 * q, 16)] * ex)

    issue_g(0, 0)

    def pair(g, _):
        for b in range(2):
            k = 2 * g + b
            wait_g(b, k)
            kp1 = k + 1
            pl.when(kp1 < NCHUNK)(lambda: issue_g(1 - b, kp1))
            pl.when(g > 0)(lambda: wait_s(b, k))
            compute(b, k)
            issue_s(b, k)
        return ()

    lax.fori_loop(0, NCHUNK // 2, pair, (), unroll=False)
    kt = NCHUNK - 1
    wait_s(0, kt)
    wait_s(1, kt)
    plsc.subcore_barrier()
    dbase = jnp.minimum(sid * DUMP, N - DUMP)
    pltpu.sync_copy(acc.at[pl.ds(dbase, DUMP)],
                    out_hbm.at[cid, pl.ds(dbase, DUMP)])


def _sc0_call(he, er, src3, dst3, cvec, z0):
    mesh = plsc.VectorSubcoreMesh(core_axis_name="c", subcore_axis_name="s")
    f = pl.kernel(
        _sc0_body,
        out_type=jax.ShapeDtypeStruct((NC, N, P0W), jnp.float32),
        mesh=mesh,
        compiler_params=_SC_PARAMS,
        scratch_types=[
            pltpu.VMEM_SHARED((N, P0W), jnp.float32),
            pltpu.VMEM((NCHUNK, CH), jnp.int32),
            pltpu.VMEM((NCHUNK, CH), jnp.int32),
            pltpu.VMEM((CH, HEW), jnp.float32),
            pltpu.VMEM((CH, HEW), jnp.float32),
            pltpu.VMEM((CH, 16), jnp.float32),
            pltpu.VMEM((CH, 16), jnp.float32),
            pltpu.VMEM((CH, P0W), jnp.float32),
            pltpu.VMEM((CH, P0W), jnp.float32),
            pltpu.VMEM((16,), jnp.float32),
            pltpu.SemaphoreType.DMA,
            pltpu.SemaphoreType.DMA,
            pltpu.SemaphoreType.DMA,
            pltpu.SemaphoreType.DMA,
        ],
    )
    return f(he, er, src3, dst3, cvec, z0)


def _sc1_body(h1_hbm, er1_hbm, src_hbm, dst_hbm, cvec_hbm, z_hbm, out_hbm,
              acc, srci, dsti, h10, h11, er0, er1, p0, p1, cvec_v,
              semg0, semg1, sems0, sems1):
    cid = lax.axis_index("c")
    sid = lax.axis_index("s")
    w = cid * NS + sid
    h1_b, er_b, p_b = (h10, h11), (er0, er1), (p0, p1)
    semg, sems = (semg0, semg1), (sems0, sems1)

    zbase = jnp.minimum(sid * DUMP, N - DUMP)
    pltpu.sync_copy(z_hbm.at[pl.ds(zbase, DUMP)], acc.at[pl.ds(zbase, DUMP)])
    pltpu.sync_copy(cvec_hbm, cvec_v)
    pltpu.sync_copy(src_hbm.at[w], srci)
    pltpu.sync_copy(dst_hbm.at[w], dsti)
    plsc.subcore_barrier()

    iot = lax.iota(jnp.int32, 16)
    zv = jnp.zeros((16,), jnp.float32)

    def issue_g(b, k):
        pltpu.async_copy(h1_hbm.at[srci.at[k]], h1_b[b], semg[b])
        pltpu.async_copy(er1_hbm.at[dsti.at[k]], er_b[b], semg[b])

    def wait_g(b, k):
        pltpu.make_async_copy(h1_hbm.at[srci.at[k]], h1_b[b], semg[b]).wait()
        pltpu.make_async_copy(er1_hbm.at[dsti.at[k]], er_b[b], semg[b]).wait()

    def issue_s(b, k):
        pltpu.async_copy(p_b[b], acc.at[dsti.at[k]], sems[b], add=True)

    def wait_s(b, k):
        pltpu.make_async_copy(
            p_b[b], acc.at[dsti.at[k]], sems[b]).wait()

    def compute(b, k):
        cv = cvec_v[...]
        cs = cv[0]
        h1_v, er1_v, p_v = h1_b[b], er_b[b], p_b[b]
        for i in range(CH):
            h3 = h1_v[i, pl.ds(32, 16)]
            erv = er1_v[i, pl.ds(0, 16)]
            e0 = h3[8] + erv[0]
            e0 = jnp.maximum(e0, NEG * e0) - cs
            exv = jnp.exp(jnp.full((16,), e0, jnp.float32))
            p_v[i, pl.ds(0, 16)] = h1_v[i, pl.ds(0, 16)] * exv
            p_v[i, pl.ds(16, 16)] = h1_v[i, pl.ds(16, 16)] * exv
            p_v[i, pl.ds(32, 16)] = jnp.where(
                iot < 8, h3 * exv, jnp.where(iot == 8, exv, zv))

    issue_g(0, 0)

    def pair(g, _):
        for b in range(2):
            k = 2 * g + b
            wait_g(b, k)
            kp1 = k + 1
            pl.when(kp1 < NCHUNK)(lambda: issue_g(1 - b, kp1))
            pl.when(g > 0)(lambda: wait_s(b, k))
            compute(b, k)
            issue_s(b, k)
        return ()

    lax.fori_loop(0, NCHUNK // 2, pair, (), unroll=False)
    kt = NCHUNK - 1
    wait_s(0, kt)
    wait_s(1, kt)
    plsc.subcore_barrier()
    dbase = jnp.minimum(sid * DUMP, N - DUMP)
    pltpu.sync_copy(acc.at[pl.ds(dbase, DUMP)],
                    out_hbm.at[cid, pl.ds(dbase, DUMP)])


def _sc1_call(h1, er1, src3, dst3, cvec, z1):
    mesh = plsc.VectorSubcoreMesh(core_axis_name="c", subcore_axis_name="s")
    f = pl.kernel(
        _sc1_body,
        out_type=jax.ShapeDtypeStruct((NC, N, P1W), jnp.float32),
        mesh=mesh,
        compiler_params=_SC_PARAMS,
        scratch_types=[
            pltpu.VMEM_SHARED((N, P1W), jnp.float32),
            pltpu.VMEM((NCHUNK, CH), jnp.int32),
            pltpu.VMEM((NCHUNK, CH), jnp.int32),
            pltpu.VMEM((CH, P1W), jnp.float32),
            pltpu.VMEM((CH, P1W), jnp.float32),
            pltpu.VMEM((CH, 16), jnp.float32),
            pltpu.VMEM((CH, 16), jnp.float32),
            pltpu.VMEM((CH, P1W), jnp.float32),
            pltpu.VMEM((CH, P1W), jnp.float32),
            pltpu.VMEM((16,), jnp.float32),
            pltpu.SemaphoreType.DMA,
            pltpu.SemaphoreType.DMA,
            pltpu.SemaphoreType.DMA,
            pltpu.SemaphoreType.DMA,
        ],
    )
    return f(h1, er1, src3, dst3, cvec, z1)


def kernel(x, edge_index, W0, attn_l0, attn_r0, b0, W1, attn_l1, attn_r1,
           b1):
    src3 = edge_index[0].reshape(NW, NCHUNK, CH)
    dst3 = edge_index[1].reshape(NW, NCHUNK, CH)

    j = jnp.arange(HD0 * F0)
    perm = (j % HD0) * F0 + j // HD0
    W0p = W0[:, perm]
    hd_of = j % HD0
    alp = jnp.zeros((HD0 * F0, HD0), jnp.float32).at[j, hd_of].set(
        attn_l0[hd_of, j // HD0])
    arp = jnp.zeros((HD0 * F0, HD0), jnp.float32).at[j, hd_of].set(
        attn_r0[hd_of, j // HD0])
    b0p = b0[perm]
    W1p = W1[perm, :]

    he, er, m0 = _k1_call(x, W0p, alp, arp)
    c0 = jnp.max(m0[:, 0, 0:8], axis=0) + jnp.max(m0[:, 0, 8:16], axis=0)
    c0 = jnp.maximum(c0, NEG * c0)
    cvec0 = jnp.concatenate([c0, c0])

    z0 = jnp.zeros((N, P0W), jnp.float32)
    acc0 = _sc0_call(he, er, src3, dst3, cvec0, z0)

    h1, er1, m1 = _k3_call(acc0, b0p, W1p, attn_l1, attn_r1)
    c1 = jnp.max(m1[:, 0, 0]) + jnp.max(m1[:, 0, 1])
    c1 = jnp.maximum(c1, NEG * c1)
    cvec1 = jnp.full((16,), c1, jnp.float32)

    z1 = jnp.zeros((N, P1W), jnp.float32)
    acc1 = _sc1_call(h1, er1, src3, dst3, cvec1, z1)

    return _k5_call(acc1, b1)


# speedup vs baseline: 1.2968x; 1.1719x over previous
import functools

import jax
import jax.numpy as jnp
from jax import lax
from jax.experimental import pallas as pl
from jax.experimental.pallas import tpu as pltpu
from jax.experimental.pallas import tpu_sc as plsc

N = 10000
E = 320000
D = 128
HD0, F0 = 8, 8
F1 = 40
NEG = 0.2

NC, NS, L = 2, 16, 16
NW = NC * NS
EPW = E // NW
CH = 80
NCHUNK = EPW // CH

RB = 2000
GRID = N // RB
NBUF = 4

HEW = 80
P0W = 80
P1W = 48
DUMP = 640

_SC_PARAMS = pltpu.CompilerParams(use_tc_tiling_on_sc=False)


def _k1_body(x_ref, w_ref, al_ref, ar_ref, he_ref, er_ref, m_ref):
    h = jnp.dot(x_ref[...], w_ref[...], preferred_element_type=jnp.float32)
    el = jnp.dot(h, al_ref[...], preferred_element_type=jnp.float32)
    er = jnp.dot(h, ar_ref[...], preferred_element_type=jnp.float32)
    he_ref[...] = jnp.concatenate([h, el, el], axis=1)
    er_ref[...] = jnp.concatenate([er, er], axis=1)
    cat = jnp.concatenate([el, er], axis=1)
    m_ref[...] = jnp.max(cat, axis=0, keepdims=True)[None]


def _k1_call(x, W0p, alp, arp):
    return pl.pallas_call(
        _k1_body,
        grid=(GRID,),
        in_specs=[
            pl.BlockSpec((RB, D), lambda i: (i, 0)),
            pl.BlockSpec((D, HD0 * F0), lambda i: (0, 0)),
            pl.BlockSpec((HD0 * F0, HD0), lambda i: (0, 0)),
            pl.BlockSpec((HD0 * F0, HD0), lambda i: (0, 0)),
        ],
        out_specs=[
            pl.BlockSpec((RB, HEW), lambda i: (i, 0)),
            pl.BlockSpec((RB, 16), lambda i: (i, 0)),
            pl.BlockSpec((1, 1, 16), lambda i: (i, 0, 0)),
        ],
        out_shape=[
            jax.ShapeDtypeStruct((N, HEW), jnp.float32),
            jax.ShapeDtypeStruct((N, 16), jnp.float32),
            jax.ShapeDtypeStruct((GRID, 1, 16), jnp.float32),
        ],
    )(x, W0p, alp, arp)


def _k3_body(a_ref, b0_ref, w1_ref, al1_ref, ar1_ref, h1_ref, er1_ref, m_ref):
    a = a_ref[...]
    t = a[0] + a[1]
    s = t[:, 0:8]
    num = t[:, 16:P0W]
    srep = jnp.broadcast_to(s[:, None, :], (RB, 8, 8)).reshape(RB, 64)
    rst = jnp.maximum(num / (srep + 1e-9) + b0_ref[...][None, :], 0.0)
    h1 = jnp.dot(rst, w1_ref[...], preferred_element_type=jnp.float32)
    el1 = jnp.sum(h1 * al1_ref[...][0][None, :], axis=1, keepdims=True)
    er1 = jnp.sum(h1 * ar1_ref[...][0][None, :], axis=1, keepdims=True)
    h1_ref[...] = jnp.concatenate(
        [h1, el1, jnp.zeros((RB, 7), jnp.float32)], axis=1)
    er1_ref[...] = jnp.concatenate(
        [er1, jnp.zeros((RB, 15), jnp.float32)], axis=1)
    m = jnp.concatenate(
        [jnp.max(el1).reshape(1), jnp.max(er1).reshape(1),
         jnp.zeros((14,), jnp.float32)])
    m_ref[...] = m.reshape(1, 1, 16)


def _k3_call(acc0, b0, W1, al1, ar1):
    return pl.pallas_call(
        _k3_body,
        grid=(GRID,),
        in_specs=[
            pl.BlockSpec((2, RB, P0W), lambda i: (0, i, 0)),
            pl.BlockSpec((HD0 * F0,), lambda i: (0,)),
            pl.BlockSpec((HD0 * F0, F1), lambda i: (0, 0)),
            pl.BlockSpec((1, F1), lambda i: (0, 0)),
            pl.BlockSpec((1, F1), lambda i: (0, 0)),
        ],
        out_specs=[
            pl.BlockSpec((RB, P1W), lambda i: (i, 0)),
            pl.BlockSpec((RB, 16), lambda i: (i, 0)),
            pl.BlockSpec((1, 1, 16), lambda i: (i, 0, 0)),
        ],
        out_shape=[
            jax.ShapeDtypeStruct((N, P1W), jnp.float32),
            jax.ShapeDtypeStruct((N, 16), jnp.float32),
            jax.ShapeDtypeStruct((GRID, 1, 16), jnp.float32),
        ],
    )(acc0, b0, W1, al1, ar1)


def _k5_body(a_ref, b1_ref, o_ref):
    a = a_ref[...]
    t = a[0] + a[1]
    num = t[:, 0:F1]
    s = t[:, F1:F1 + 1]
    o_ref[...] = num / (s + 1e-9) + b1_ref[...][None, :]


def _k5_call(acc1, b1):
    return pl.pallas_call(
        _k5_body,
        grid=(GRID,),
        in_specs=[
            pl.BlockSpec((2, RB, P1W), lambda i: (0, i, 0)),
            pl.BlockSpec((F1,), lambda i: (0,)),
        ],
        out_specs=pl.BlockSpec((RB, F1), lambda i: (i, 0)),
        out_shape=jax.ShapeDtypeStruct((N, F1), jnp.float32),
    )(acc1, b1)


def _sc0_body(he_hbm, er_hbm, src_hbm, dst_hbm, cvec_hbm, z_hbm, out_hbm,
              acc, srci, dsti, he0, he1, he2, he3, er0, er1, er2, er3,
              p0, p1, p2, p3, cvec_v,
              semg0, semg1, semg2, semg3, sems0, sems1, sems2, sems3):
    cid = lax.axis_index("c")
    sid = lax.axis_index("s")
    w = cid * NS + sid
    he_b, er_b, p_b = (he0, he1, he2, he3), (er0, er1, er2, er3), \
        (p0, p1, p2, p3)
    semg, sems = (semg0, semg1, semg2, semg3), (sems0, sems1, sems2, sems3)

    zbase = jnp.minimum(sid * DUMP, N - DUMP)
    pltpu.sync_copy(z_hbm.at[pl.ds(zbase, DUMP)], acc.at[pl.ds(zbase, DUMP)])
    pltpu.sync_copy(cvec_hbm, cvec_v)
    pltpu.sync_copy(src_hbm.at[w], srci)
    pltpu.sync_copy(dst_hbm.at[w], dsti)
    plsc.subcore_barrier()

    def issue_g(b, k):
        pltpu.async_copy(he_hbm.at[srci.at[k]], he_b[b], semg[b])
        pltpu.async_copy(er_hbm.at[dsti.at[k]], er_b[b], semg[b])

    def wait_g(b, k):
        pltpu.make_async_copy(he_hbm.at[srci.at[k]], he_b[b], semg[b]).wait()
        pltpu.make_async_copy(er_hbm.at[dsti.at[k]], er_b[b], semg[b]).wait()

    def issue_s(b, k):
        pltpu.async_copy(p_b[b], acc.at[dsti.at[k]], sems[b], add=True)

    def wait_s(b, k):
        pltpu.make_async_copy(
            p_b[b], acc.at[dsti.at[k]], sems[b]).wait()

    def compute(b, k):
        cv = cvec_v[...]
        he_v, er_v, p_v = he_b[b], er_b[b], p_b[b]
        for i in range(CH):
            el = he_v[i, pl.ds(64, 16)]
            er = er_v[i, pl.ds(0, 16)]
            e = el + er
            e = jnp.maximum(e, NEG * e)
            ex = jnp.exp(e - cv)
            p_v[i, pl.ds(0, 16)] = ex
            for q in range(4):
                p_v[i, pl.ds(16 + 16 * q, 16)] = (
                    he_v[i, pl.ds(16 * q, 16)] * ex)

    for kk in range(NBUF - 1):
        issue_g(kk, kk)

    def quad(g, _):
        for b in range(NBUF):
            k = NBUF * g + b
            wait_g(b, k)
            kp = k + NBUF - 1
            pl.when(kp < NCHUNK)(lambda: issue_g((b + NBUF - 1) % NBUF, kp))
            pl.when(g > 0)(lambda: wait_s(b, k))
            compute(b, k)
            issue_s(b, k)
        return ()

    lax.fori_loop(0, NCHUNK // NBUF, quad, (), unroll=False)
    kt = NCHUNK - 1
    for t in range((NCHUNK // NBUF) * NBUF, NCHUNK):
        b = t % NBUF
        wait_g(b, t)
        wait_s(b, t)
        compute(b, t)
        issue_s(b, t)
    for b in range(NBUF):
        wait_s(b, kt)
    plsc.subcore_barrier()
    dbase = jnp.minimum(sid * DUMP, N - DUMP)
    pltpu.sync_copy(acc.at[pl.ds(dbase, DUMP)],
                    out_hbm.at[cid, pl.ds(dbase, DUMP)])


def _sc0_call(he, er, src3, dst3, cvec, z0):
    mesh = plsc.VectorSubcoreMesh(core_axis_name="c", subcore_axis_name="s")
    f = pl.kernel(
        _sc0_body,
        out_type=jax.ShapeDtypeStruct((NC, N, P0W), jnp.float32),
        mesh=mesh,
        compiler_params=_SC_PARAMS,
        scratch_types=[
            pltpu.VMEM_SHARED((N, P0W), jnp.float32),
            pltpu.VMEM((NCHUNK, CH), jnp.int32),
            pltpu.VMEM((NCHUNK, CH), jnp.int32),
        ] + [pltpu.VMEM((CH, HEW), jnp.float32)] * NBUF
        + [pltpu.VMEM((CH, 16), jnp.float32)] * NBUF
        + [pltpu.VMEM((CH, P0W), jnp.float32)] * NBUF
        + [pltpu.VMEM((16,), jnp.float32)]
        + [pltpu.SemaphoreType.DMA] * (2 * NBUF),
    )
    return f(he, er, src3, dst3, cvec, z0)


def _sc1_body(h1_hbm, er1_hbm, src_hbm, dst_hbm, cvec_hbm, z_hbm, out_hbm,
              acc, srci, dsti, h10, h11, h12, h13, er0, er1, er2, er3,
              p0, p1, p2, p3, cvec_v,
              semg0, semg1, semg2, semg3, sems0, sems1, sems2, sems3):
    cid = lax.axis_index("c")
    sid = lax.axis_index("s")
    w = cid * NS + sid
    h1_b, er_b, p_b = (h10, h11, h12, h13), (er0, er1, er2, er3), \
        (p0, p1, p2, p3)
    semg, sems = (semg0, semg1, semg2, semg3), (sems0, sems1, sems2, sems3)

    zbase = jnp.minimum(sid * DUMP, N - DUMP)
    pltpu.sync_copy(z_hbm.at[pl.ds(zbase, DUMP)], acc.at[pl.ds(zbase, DUMP)])
    pltpu.sync_copy(cvec_hbm, cvec_v)
    pltpu.sync_copy(src_hbm.at[w], srci)
    pltpu.sync_copy(dst_hbm.at[w], dsti)
    plsc.subcore_barrier()

    iot = lax.iota(jnp.int32, 16)
    zv = jnp.zeros((16,), jnp.float32)

    def issue_g(b, k):
        pltpu.async_copy(h1_hbm.at[srci.at[k]], h1_b[b], semg[b])
        pltpu.async_copy(er1_hbm.at[dsti.at[k]], er_b[b], semg[b])

    def wait_g(b, k):
        pltpu.make_async_copy(h1_hbm.at[srci.at[k]], h1_b[b], semg[b]).wait()
        pltpu.make_async_copy(er1_hbm.at[dsti.at[k]], er_b[b], semg[b]).wait()

    def issue_s(b, k):
        pltpu.async_copy(p_b[b], acc.at[dsti.at[k]], sems[b], add=True)

    def wait_s(b, k):
        pltpu.make_async_copy(
            p_b[b], acc.at[dsti.at[k]], sems[b]).wait()

    def compute(b, k):
        cv = cvec_v[...]
        cs = cv[0]
        h1_v, er1_v, p_v = h1_b[b], er_b[b], p_b[b]
        for i in range(CH):
            h3 = h1_v[i, pl.ds(32, 16)]
            erv = er1_v[i, pl.ds(0, 16)]
            e0 = h3[8] + erv[0]
            e0 = jnp.maximum(e0, NEG * e0) - cs
            exv = jnp.exp(jnp.full((16,), e0, jnp.float32))
            p_v[i, pl.ds(0, 16)] = h1_v[i, pl.ds(0, 16)] * exv
            p_v[i, pl.ds(16, 16)] = h1_v[i, pl.ds(16, 16)] * exv
            p_v[i, pl.ds(32, 16)] = jnp.where(
                iot < 8, h3 * exv, jnp.where(iot == 8, exv, zv))

    for kk in range(NBUF - 1):
        issue_g(kk, kk)

    def quad(g, _):
        for b in range(NBUF):
            k = NBUF * g + b
            wait_g(b, k)
            kp = k + NBUF - 1
            pl.when(kp < NCHUNK)(lambda: issue_g((b + NBUF - 1) % NBUF, kp))
            pl.when(g > 0)(lambda: wait_s(b, k))
            compute(b, k)
            issue_s(b, k)
        return ()

    lax.fori_loop(0, NCHUNK // NBUF, quad, (), unroll=False)
    kt = NCHUNK - 1
    for t in range((NCHUNK // NBUF) * NBUF, NCHUNK):
        b = t % NBUF
        wait_g(b, t)
        wait_s(b, t)
        compute(b, t)
        issue_s(b, t)
    for b in range(NBUF):
        wait_s(b, kt)
    plsc.subcore_barrier()
    dbase = jnp.minimum(sid * DUMP, N - DUMP)
    pltpu.sync_copy(acc.at[pl.ds(dbase, DUMP)],
                    out_hbm.at[cid, pl.ds(dbase, DUMP)])


def _sc1_call(h1, er1, src3, dst3, cvec, z1):
    mesh = plsc.VectorSubcoreMesh(core_axis_name="c", subcore_axis_name="s")
    f = pl.kernel(
        _sc1_body,
        out_type=jax.ShapeDtypeStruct((NC, N, P1W), jnp.float32),
        mesh=mesh,
        compiler_params=_SC_PARAMS,
        scratch_types=[
            pltpu.VMEM_SHARED((N, P1W), jnp.float32),
            pltpu.VMEM((NCHUNK, CH), jnp.int32),
            pltpu.VMEM((NCHUNK, CH), jnp.int32),
        ] + [pltpu.VMEM((CH, P1W), jnp.float32)] * NBUF
        + [pltpu.VMEM((CH, 16), jnp.float32)] * NBUF
        + [pltpu.VMEM((CH, P1W), jnp.float32)] * NBUF
        + [pltpu.VMEM((16,), jnp.float32)]
        + [pltpu.SemaphoreType.DMA] * (2 * NBUF),
    )
    return f(h1, er1, src3, dst3, cvec, z1)


def kernel(x, edge_index, W0, attn_l0, attn_r0, b0, W1, attn_l1, attn_r1,
           b1):
    src3 = edge_index[0].reshape(NW, NCHUNK, CH)
    dst3 = edge_index[1].reshape(NW, NCHUNK, CH)

    j = jnp.arange(HD0 * F0)
    perm = (j % HD0) * F0 + j // HD0
    W0p = W0[:, perm]
    hd_of = j % HD0
    alp = jnp.zeros((HD0 * F0, HD0), jnp.float32).at[j, hd_of].set(
        attn_l0[hd_of, j // HD0])
    arp = jnp.zeros((HD0 * F0, HD0), jnp.float32).at[j, hd_of].set(
        attn_r0[hd_of, j // HD0])
    b0p = b0[perm]
    W1p = W1[perm, :]

    he, er, m0 = _k1_call(x, W0p, alp, arp)
    c0 = jnp.max(m0[:, 0, 0:8], axis=0) + jnp.max(m0[:, 0, 8:16], axis=0)
    c0 = jnp.maximum(c0, NEG * c0)
    cvec0 = jnp.concatenate([c0, c0])

    z0 = jnp.zeros((N, P0W), jnp.float32)
    acc0 = _sc0_call(he, er, src3, dst3, cvec0, z0)

    h1, er1, m1 = _k3_call(acc0, b0p, W1p, attn_l1, attn_r1)
    c1 = jnp.max(m1[:, 0, 0]) + jnp.max(m1[:, 0, 1])
    c1 = jnp.maximum(c1, NEG * c1)
    cvec1 = jnp.full((16,), c1, jnp.float32)

    z1 = jnp.zeros((N, P1W), jnp.float32)
    acc1 = _sc1_call(h1, er1, src3, dst3, cvec1, z1)

    return _k5_call(acc1, b1)


# speedup vs baseline: 1.3453x; 1.0374x over previous
import functools

import jax
import jax.numpy as jnp
from jax import lax
from jax.experimental import pallas as pl
from jax.experimental.pallas import tpu as pltpu
from jax.experimental.pallas import tpu_sc as plsc

N = 10000
E = 320000
D = 128
HD0, F0 = 8, 8
F1 = 40
NEG = 0.2

NC, NS, L = 2, 16, 16
NW = NC * NS
EPW = E // NW
CH = 80
NCHUNK = EPW // CH

RB = 2000
GRID = N // RB
NBUF = 4

HEW = 80
P0W = 72
P1W = 48
DUMP = 640

_SC_PARAMS = pltpu.CompilerParams(use_tc_tiling_on_sc=False)


def _perm_consts():
    n = HD0 * F0
    ri = lax.broadcasted_iota(jnp.int32, (n, n), 0)
    ci = lax.broadcasted_iota(jnp.int32, (n, n), 1)
    S = (ri == (ci % HD0) * F0 + ci // HD0).astype(jnp.float32)
    T = ((ri % HD0) * F0 + ri // HD0 == ci).astype(jnp.float32)
    rj = lax.broadcasted_iota(jnp.int32, (n, HD0), 0)
    cj = lax.broadcasted_iota(jnp.int32, (n, HD0), 1)
    mask = (rj % HD0) == cj
    return S, T, mask


def _k1_body(x_ref, w_ref, al_ref, ar_ref, he_ref, er_ref, m_ref):
    S, _, mask = _perm_consts()
    h0 = jnp.dot(x_ref[...], w_ref[...], preferred_element_type=jnp.float32)
    h = jnp.dot(h0, S, preferred_element_type=jnp.float32)
    alT = al_ref[...].T
    alp = jnp.where(mask, jnp.broadcast_to(
        alT[:, None, :], (F0, HD0, HD0)).reshape(HD0 * F0, HD0), 0.0)
    arT = ar_ref[...].T
    arp = jnp.where(mask, jnp.broadcast_to(
        arT[:, None, :], (F0, HD0, HD0)).reshape(HD0 * F0, HD0), 0.0)
    el = jnp.dot(h, alp, preferred_element_type=jnp.float32)
    er = jnp.dot(h, arp, preferred_element_type=jnp.float32)
    he_ref[...] = jnp.concatenate([h, el, el], axis=1)
    er_ref[...] = jnp.concatenate([er, er], axis=1)
    cat = jnp.concatenate([el, er], axis=1)
    m_ref[...] = jnp.max(cat, axis=0, keepdims=True)[None]


def _k1_call(x, W0, al0, ar0):
    return pl.pallas_call(
        _k1_body,
        grid=(GRID,),
        in_specs=[
            pl.BlockSpec((RB, D), lambda i: (i, 0)),
            pl.BlockSpec((D, HD0 * F0), lambda i: (0, 0)),
            pl.BlockSpec((HD0, F0), lambda i: (0, 0)),
            pl.BlockSpec((HD0, F0), lambda i: (0, 0)),
        ],
        out_specs=[
            pl.BlockSpec((RB, HEW), lambda i: (i, 0)),
            pl.BlockSpec((RB, 16), lambda i: (i, 0)),
            pl.BlockSpec((1, 1, 16), lambda i: (i, 0, 0)),
        ],
        out_shape=[
            jax.ShapeDtypeStruct((N, HEW), jnp.float32),
            jax.ShapeDtypeStruct((N, 16), jnp.float32),
            jax.ShapeDtypeStruct((GRID, 1, 16), jnp.float32),
        ],
    )(x, W0, al0, ar0)


def _k3_body(a_ref, b0_ref, w1_ref, al1_ref, ar1_ref, h1_ref, er1_ref, m_ref):
    a = a_ref[...]
    t = a[0] + a[1]
    S, T, _ = _perm_consts()
    s = t[:, 0:8]
    num = t[:, 8:P0W]
    srep = jnp.broadcast_to(s[:, None, :], (RB, 8, 8)).reshape(RB, 64)
    b0p = jnp.dot(b0_ref[...][None, :], S,
                  preferred_element_type=jnp.float32)
    rst = jnp.maximum(num / (srep + 1e-9) + b0p, 0.0)
    rst = jnp.dot(rst, T, preferred_element_type=jnp.float32)
    h1 = jnp.dot(rst, w1_ref[...], preferred_element_type=jnp.float32)
    el1 = jnp.sum(h1 * al1_ref[...][0][None, :], axis=1, keepdims=True)
    er1 = jnp.sum(h1 * ar1_ref[...][0][None, :], axis=1, keepdims=True)
    h1_ref[...] = jnp.concatenate(
        [h1, el1, jnp.zeros((RB, 7), jnp.float32)], axis=1)
    er1_ref[...] = jnp.concatenate(
        [er1, jnp.zeros((RB, 15), jnp.float32)], axis=1)
    m = jnp.concatenate(
        [jnp.max(el1).reshape(1), jnp.max(er1).reshape(1),
         jnp.zeros((14,), jnp.float32)])
    m_ref[...] = m.reshape(1, 1, 16)


def _k3_call(acc0, b0, W1, al1, ar1):
    return pl.pallas_call(
        _k3_body,
        grid=(GRID,),
        in_specs=[
            pl.BlockSpec((2, RB, P0W), lambda i: (0, i, 0)),
            pl.BlockSpec((HD0 * F0,), lambda i: (0,)),
            pl.BlockSpec((HD0 * F0, F1), lambda i: (0, 0)),
            pl.BlockSpec((1, F1), lambda i: (0, 0)),
            pl.BlockSpec((1, F1), lambda i: (0, 0)),
        ],
        out_specs=[
            pl.BlockSpec((RB, P1W), lambda i: (i, 0)),
            pl.BlockSpec((RB, 16), lambda i: (i, 0)),
            pl.BlockSpec((1, 1, 16), lambda i: (i, 0, 0)),
        ],
        out_shape=[
            jax.ShapeDtypeStruct((N, P1W), jnp.float32),
            jax.ShapeDtypeStruct((N, 16), jnp.float32),
            jax.ShapeDtypeStruct((GRID, 1, 16), jnp.float32),
        ],
    )(acc0, b0, W1, al1, ar1)


def _k5_body(a_ref, b1_ref, o_ref):
    a = a_ref[...]
    t = a[0] + a[1]
    num = t[:, 0:F1]
    s = t[:, F1:F1 + 1]
    o_ref[...] = num / (s + 1e-9) + b1_ref[...][None, :]


def _k5_call(acc1, b1):
    return pl.pallas_call(
        _k5_body,
        grid=(GRID,),
        in_specs=[
            pl.BlockSpec((2, RB, P1W), lambda i: (0, i, 0)),
            pl.BlockSpec((F1,), lambda i: (0,)),
        ],
        out_specs=pl.BlockSpec((RB, F1), lambda i: (i, 0)),
        out_shape=jax.ShapeDtypeStruct((N, F1), jnp.float32),
    )(acc1, b1)


def _sc0_body(he_hbm, er_hbm, src_hbm, dst_hbm, cvec_hbm, z_hbm, out_hbm,
              acc, srci, dsti, he0, he1, he2, he3, er0, er1, er2, er3,
              p0, p1, p2, p3, cvec_v,
              semg0, semg1, semg2, semg3, sems0, sems1, sems2, sems3):
    cid = lax.axis_index("c")
    sid = lax.axis_index("s")
    w = cid * NS + sid
    he_b, er_b, p_b = (he0, he1, he2, he3), (er0, er1, er2, er3), \
        (p0, p1, p2, p3)
    semg, sems = (semg0, semg1, semg2, semg3), (sems0, sems1, sems2, sems3)

    zbase = jnp.minimum(sid * DUMP, N - DUMP)
    pltpu.sync_copy(z_hbm.at[pl.ds(zbase, DUMP)], acc.at[pl.ds(zbase, DUMP)])
    pltpu.sync_copy(cvec_hbm, cvec_v)
    pltpu.sync_copy(src_hbm.at[w], srci)
    pltpu.sync_copy(dst_hbm.at[w], dsti)
    plsc.subcore_barrier()

    def issue_g(b, k):
        pltpu.async_copy(he_hbm.at[srci.at[k]], he_b[b], semg[b])
        pltpu.async_copy(er_hbm.at[dsti.at[k]], er_b[b], semg[b])

    def wait_g(b, k):
        pltpu.make_async_copy(he_hbm.at[srci.at[k]], he_b[b], semg[b]).wait()
        pltpu.make_async_copy(er_hbm.at[dsti.at[k]], er_b[b], semg[b]).wait()

    def issue_s(b, k):
        pltpu.async_copy(p_b[b], acc.at[dsti.at[k]], sems[b], add=True)

    def wait_s(b, k):
        pltpu.make_async_copy(
            p_b[b], acc.at[dsti.at[k]], sems[b]).wait()

    def compute(b, k):
        cv = cvec_v[...]
        he_v, er_v, p_v = he_b[b], er_b[b], p_b[b]
        for i in range(CH):
            el = he_v[i, pl.ds(64, 16)]
            er = er_v[i, pl.ds(0, 16)]
            e = el + er
            e = jnp.maximum(e, NEG * e)
            ex = jnp.exp(e - cv)
            p_v[i, pl.ds(0, 16)] = ex
            for q in range(4):
                p_v[i, pl.ds(8 + 16 * q, 16)] = (
                    he_v[i, pl.ds(16 * q, 16)] * ex)

    for kk in range(NBUF - 1):
        issue_g(kk, kk)

    def quad(g, _):
        for b in range(NBUF):
            k = NBUF * g + b
            wait_g(b, k)
            kp = k + NBUF - 1
            pl.when(kp < NCHUNK)(lambda: issue_g((b + NBUF - 1) % NBUF, kp))
            pl.when(g > 0)(lambda: wait_s(b, k))
            compute(b, k)
            issue_s(b, k)
        return ()

    lax.fori_loop(0, NCHUNK // NBUF, quad, (), unroll=False)
    kt = NCHUNK - 1
    for t in range((NCHUNK // NBUF) * NBUF, NCHUNK):
        b = t % NBUF
        wait_g(b, t)
        wait_s(b, t)
        compute(b, t)
        issue_s(b, t)
    for b in range(NBUF):
        wait_s(b, kt)
    plsc.subcore_barrier()
    dbase = jnp.minimum(sid * DUMP, N - DUMP)
    pltpu.sync_copy(acc.at[pl.ds(dbase, DUMP)],
                    out_hbm.at[cid, pl.ds(dbase, DUMP)])


def _sc0_call(he, er, src3, dst3, cvec, z0):
    mesh = plsc.VectorSubcoreMesh(core_axis_name="c", subcore_axis_name="s")
    f = pl.kernel(
        _sc0_body,
        out_type=jax.ShapeDtypeStruct((NC, N, P0W), jnp.float32),
        mesh=mesh,
        compiler_params=_SC_PARAMS,
        scratch_types=[
            pltpu.VMEM_SHARED((N, P0W), jnp.float32),
            pltpu.VMEM((NCHUNK, CH), jnp.int32),
            pltpu.VMEM((NCHUNK, CH), jnp.int32),
        ] + [pltpu.VMEM((CH, HEW), jnp.float32)] * NBUF
        + [pltpu.VMEM((CH, 16), jnp.float32)] * NBUF
        + [pltpu.VMEM((CH, P0W), jnp.float32)] * NBUF
        + [pltpu.VMEM((16,), jnp.float32)]
        + [pltpu.SemaphoreType.DMA] * (2 * NBUF),
    )
    return f(he, er, src3, dst3, cvec, z0)


def _sc1_body(h1_hbm, er1_hbm, src_hbm, dst_hbm, cvec_hbm, z_hbm, out_hbm,
              acc, srci, dsti, h10, h11, h12, h13, er0, er1, er2, er3,
              p0, p1, p2, p3, cvec_v,
              semg0, semg1, semg2, semg3, sems0, sems1, sems2, sems3):
    cid = lax.axis_index("c")
    sid = lax.axis_index("s")
    w = cid * NS + sid
    h1_b, er_b, p_b = (h10, h11, h12, h13), (er0, er1, er2, er3), \
        (p0, p1, p2, p3)
    semg, sems = (semg0, semg1, semg2, semg3), (sems0, sems1, sems2, sems3)

    zbase = jnp.minimum(sid * DUMP, N - DUMP)
    pltpu.sync_copy(z_hbm.at[pl.ds(zbase, DUMP)], acc.at[pl.ds(zbase, DUMP)])
    pltpu.sync_copy(cvec_hbm, cvec_v)
    pltpu.sync_copy(src_hbm.at[w], srci)
    pltpu.sync_copy(dst_hbm.at[w], dsti)
    plsc.subcore_barrier()

    iot = lax.iota(jnp.int32, 16)
    zv = jnp.zeros((16,), jnp.float32)

    def issue_g(b, k):
        pltpu.async_copy(h1_hbm.at[srci.at[k]], h1_b[b], semg[b])
        pltpu.async_copy(er1_hbm.at[dsti.at[k]], er_b[b], semg[b])

    def wait_g(b, k):
        pltpu.make_async_copy(h1_hbm.at[srci.at[k]], h1_b[b], semg[b]).wait()
        pltpu.make_async_copy(er1_hbm.at[dsti.at[k]], er_b[b], semg[b]).wait()

    def issue_s(b, k):
        pltpu.async_copy(p_b[b], acc.at[dsti.at[k]], sems[b], add=True)

    def wait_s(b, k):
        pltpu.make_async_copy(
            p_b[b], acc.at[dsti.at[k]], sems[b]).wait()

    def compute(b, k):
        cv = cvec_v[...]
        cs = cv[0]
        h1_v, er1_v, p_v = h1_b[b], er_b[b], p_b[b]
        for i in range(CH):
            h3 = h1_v[i, pl.ds(32, 16)]
            erv = er1_v[i, pl.ds(0, 16)]
            e0 = h3[8] + erv[0]
            e0 = jnp.maximum(e0, NEG * e0) - cs
            exv = jnp.exp(jnp.full((16,), e0, jnp.float32))
            p_v[i, pl.ds(0, 16)] = h1_v[i, pl.ds(0, 16)] * exv
            p_v[i, pl.ds(16, 16)] = h1_v[i, pl.ds(16, 16)] * exv
            p_v[i, pl.ds(32, 16)] = jnp.where(
                iot < 8, h3 * exv, jnp.where(iot == 8, exv, zv))

    for kk in range(NBUF - 1):
        issue_g(kk, kk)

    def quad(g, _):
        for b in range(NBUF):
            k = NBUF * g + b
            wait_g(b, k)
            kp = k + NBUF - 1
            pl.when(kp < NCHUNK)(lambda: issue_g((b + NBUF - 1) % NBUF, kp))
            pl.when(g > 0)(lambda: wait_s(b, k))
            compute(b, k)
            issue_s(b, k)
        return ()

    lax.fori_loop(0, NCHUNK // NBUF, quad, (), unroll=False)
    kt = NCHUNK - 1
    for t in range((NCHUNK // NBUF) * NBUF, NCHUNK):
        b = t % NBUF
        wait_g(b, t)
        wait_s(b, t)
        compute(b, t)
        issue_s(b, t)
    for b in range(NBUF):
        wait_s(b, kt)
    plsc.subcore_barrier()
    dbase = jnp.minimum(sid * DUMP, N - DUMP)
    pltpu.sync_copy(acc.at[pl.ds(dbase, DUMP)],
                    out_hbm.at[cid, pl.ds(dbase, DUMP)])


def _sc1_call(h1, er1, src3, dst3, cvec, z1):
    mesh = plsc.VectorSubcoreMesh(core_axis_name="c", subcore_axis_name="s")
    f = pl.kernel(
        _sc1_body,
        out_type=jax.ShapeDtypeStruct((NC, N, P1W), jnp.float32),
        mesh=mesh,
        compiler_params=_SC_PARAMS,
        scratch_types=[
            pltpu.VMEM_SHARED((N, P1W), jnp.float32),
            pltpu.VMEM((NCHUNK, CH), jnp.int32),
            pltpu.VMEM((NCHUNK, CH), jnp.int32),
        ] + [pltpu.VMEM((CH, P1W), jnp.float32)] * NBUF
        + [pltpu.VMEM((CH, 16), jnp.float32)] * NBUF
        + [pltpu.VMEM((CH, P1W), jnp.float32)] * NBUF
        + [pltpu.VMEM((16,), jnp.float32)]
        + [pltpu.SemaphoreType.DMA] * (2 * NBUF),
    )
    return f(h1, er1, src3, dst3, cvec, z1)


def kernel(x, edge_index, W0, attn_l0, attn_r0, b0, W1, attn_l1, attn_r1,
           b1):
    src3 = edge_index[0].reshape(NW, NCHUNK, CH)
    dst3 = edge_index[1].reshape(NW, NCHUNK, CH)

    he, er, m0 = _k1_call(x, W0, attn_l0, attn_r0)
    c0 = jnp.max(m0[:, 0, 0:8], axis=0) + jnp.max(m0[:, 0, 8:16], axis=0)
    c0 = jnp.maximum(c0, NEG * c0)
    cvec0 = jnp.concatenate([c0, c0])

    z0 = jnp.zeros((N, P0W), jnp.float32)
    acc0 = _sc0_call(he, er, src3, dst3, cvec0, z0)

    h1, er1, m1 = _k3_call(acc0, b0, W1, attn_l1, attn_r1)
    c1 = jnp.max(m1[:, 0, 0]) + jnp.max(m1[:, 0, 1])
    c1 = jnp.maximum(c1, NEG * c1)
    cvec1 = jnp.full((16,), c1, jnp.float32)

    z1 = jnp.zeros((N, P1W), jnp.float32)
    acc1 = _sc1_call(h1, er1, src3, dst3, cvec1, z1)

    return _k5_call(acc1, b1)


# speedup vs baseline: 1.3467x; 1.0011x over previous
import functools

import jax
import jax.numpy as jnp
from jax import lax
from jax.experimental import pallas as pl
from jax.experimental.pallas import tpu as pltpu
from jax.experimental.pallas import tpu_sc as plsc

N = 10000
E = 320000
D = 128
HD0, F0 = 8, 8
F1 = 40
NEG = 0.2

NC, NS, L = 2, 16, 16
NW = NC * NS
EPW = E // NW
CH = 80
NCHUNK = EPW // CH
CH1 = 125
NCHUNK1 = EPW // CH1

RB = 2000
GRID = N // RB
NBUF = 4

HEW = 80
P0W = 72
P1W = 48
DUMP = 640

_SC_PARAMS = pltpu.CompilerParams(use_tc_tiling_on_sc=False)


def _perm_consts():
    n = HD0 * F0
    ri = lax.broadcasted_iota(jnp.int32, (n, n), 0)
    ci = lax.broadcasted_iota(jnp.int32, (n, n), 1)
    S = (ri == (ci % HD0) * F0 + ci // HD0).astype(jnp.float32)
    T = ((ri % HD0) * F0 + ri // HD0 == ci).astype(jnp.float32)
    rj = lax.broadcasted_iota(jnp.int32, (n, HD0), 0)
    cj = lax.broadcasted_iota(jnp.int32, (n, HD0), 1)
    mask = (rj % HD0) == cj
    return S, T, mask


def _k1_body(x_ref, w_ref, al_ref, ar_ref, he_ref, er_ref, cv_ref, macc_ref):
    S, _, mask = _perm_consts()
    h0 = jnp.dot(x_ref[...], w_ref[...], preferred_element_type=jnp.float32)
    h = jnp.dot(h0, S, preferred_element_type=jnp.float32)
    alT = al_ref[...].T
    alp = jnp.where(mask, jnp.broadcast_to(
        alT[:, None, :], (F0, HD0, HD0)).reshape(HD0 * F0, HD0), 0.0)
    arT = ar_ref[...].T
    arp = jnp.where(mask, jnp.broadcast_to(
        arT[:, None, :], (F0, HD0, HD0)).reshape(HD0 * F0, HD0), 0.0)
    el = jnp.dot(h, alp, preferred_element_type=jnp.float32)
    er = jnp.dot(h, arp, preferred_element_type=jnp.float32)
    he_ref[...] = jnp.concatenate([h, el, el], axis=1)
    er_ref[...] = jnp.concatenate([er, er], axis=1)
    cat = jnp.concatenate([el, er], axis=1)
    cm = jnp.max(cat, axis=0, keepdims=True)
    i = pl.program_id(0)

    @pl.when(i == 0)
    def _():
        macc_ref[...] = cm

    @pl.when(i != 0)
    def _():
        macc_ref[...] = jnp.maximum(macc_ref[...], cm)

    @pl.when(i == GRID - 1)
    def _():
        m = macc_ref[...]
        c8 = m[:, 0:8] + m[:, 8:16]
        c8 = jnp.maximum(c8, NEG * c8)
        cv_ref[...] = jnp.concatenate([c8, c8], axis=1)


def _k1_call(x, W0, al0, ar0):
    return pl.pallas_call(
        _k1_body,
        grid=(GRID,),
        in_specs=[
            pl.BlockSpec((RB, D), lambda i: (i, 0)),
            pl.BlockSpec((D, HD0 * F0), lambda i: (0, 0)),
            pl.BlockSpec((HD0, F0), lambda i: (0, 0)),
            pl.BlockSpec((HD0, F0), lambda i: (0, 0)),
        ],
        out_specs=[
            pl.BlockSpec((RB, HEW), lambda i: (i, 0)),
            pl.BlockSpec((RB, 16), lambda i: (i, 0)),
            pl.BlockSpec((1, 16), lambda i: (0, 0)),
        ],
        out_shape=[
            jax.ShapeDtypeStruct((N, HEW), jnp.float32),
            jax.ShapeDtypeStruct((N, 16), jnp.float32),
            jax.ShapeDtypeStruct((1, 16), jnp.float32),
        ],
        scratch_shapes=[pltpu.VMEM((1, 16), jnp.float32)],
    )(x, W0, al0, ar0)


def _k3_body(a_ref, b0_ref, w1_ref, al1_ref, ar1_ref, h1_ref, er1_ref,
             cv_ref, macc_ref):
    a = a_ref[...]
    t = a[0] + a[1]
    S, T, _ = _perm_consts()
    s = t[:, 0:8]
    num = t[:, 8:P0W]
    srep = jnp.broadcast_to(s[:, None, :], (RB, 8, 8)).reshape(RB, 64)
    b0p = jnp.dot(b0_ref[...][None, :], S,
                  preferred_element_type=jnp.float32)
    rst = jnp.maximum(num / (srep + 1e-9) + b0p, 0.0)
    rst = jnp.dot(rst, T, preferred_element_type=jnp.float32)
    h1 = jnp.dot(rst, w1_ref[...], preferred_element_type=jnp.float32)
    el1 = jnp.sum(h1 * al1_ref[...][0][None, :], axis=1, keepdims=True)
    er1 = jnp.sum(h1 * ar1_ref[...][0][None, :], axis=1, keepdims=True)
    h1_ref[...] = jnp.concatenate(
        [h1, el1, jnp.zeros((RB, 7), jnp.float32)], axis=1)
    er1_ref[...] = jnp.concatenate(
        [er1, jnp.zeros((RB, 15), jnp.float32)], axis=1)
    cm = jnp.concatenate(
        [jnp.max(el1).reshape(1, 1), jnp.max(er1).reshape(1, 1),
         jnp.full((1, 14), -1e30, jnp.float32)], axis=1)
    i = pl.program_id(0)

    @pl.when(i == 0)
    def _():
        macc_ref[...] = cm

    @pl.when(i != 0)
    def _():
        macc_ref[...] = jnp.maximum(macc_ref[...], cm)

    @pl.when(i == GRID - 1)
    def _():
        m = macc_ref[...]
        c1 = m[0, 0] + m[0, 1]
        c1 = jnp.maximum(c1, NEG * c1)
        cv_ref[...] = jnp.full((1, 16), c1, jnp.float32)


def _k3_call(acc0, b0, W1, al1, ar1):
    return pl.pallas_call(
        _k3_body,
        grid=(GRID,),
        in_specs=[
            pl.BlockSpec((2, RB, P0W), lambda i: (0, i, 0)),
            pl.BlockSpec((HD0 * F0,), lambda i: (0,)),
            pl.BlockSpec((HD0 * F0, F1), lambda i: (0, 0)),
            pl.BlockSpec((1, F1), lambda i: (0, 0)),
            pl.BlockSpec((1, F1), lambda i: (0, 0)),
        ],
        out_specs=[
            pl.BlockSpec((RB, P1W), lambda i: (i, 0)),
            pl.BlockSpec((RB, 16), lambda i: (i, 0)),
            pl.BlockSpec((1, 16), lambda i: (0, 0)),
        ],
        out_shape=[
            jax.ShapeDtypeStruct((N, P1W), jnp.float32),
            jax.ShapeDtypeStruct((N, 16), jnp.float32),
            jax.ShapeDtypeStruct((1, 16), jnp.float32),
        ],
        scratch_shapes=[pltpu.VMEM((1, 16), jnp.float32)],
    )(acc0, b0, W1, al1, ar1)


def _k5_body(a_ref, b1_ref, o_ref):
    a = a_ref[...]
    t = a[0] + a[1]
    num = t[:, 0:F1]
    s = t[:, F1:F1 + 1]
    o_ref[...] = num / (s + 1e-9) + b1_ref[...][None, :]


def _k5_call(acc1, b1):
    return pl.pallas_call(
        _k5_body,
        grid=(GRID,),
        in_specs=[
            pl.BlockSpec((2, RB, P1W), lambda i: (0, i, 0)),
            pl.BlockSpec((F1,), lambda i: (0,)),
        ],
        out_specs=pl.BlockSpec((RB, F1), lambda i: (i, 0)),
        out_shape=jax.ShapeDtypeStruct((N, F1), jnp.float32),
    )(acc1, b1)


def _sc0_body(he_hbm, er_hbm, src_hbm, dst_hbm, cvec_hbm, z_hbm, out_hbm,
              acc, srci, dsti, he0, he1, he2, he3, er0, er1, er2, er3,
              p0, p1, p2, p3, cvec_v,
              semg0, semg1, semg2, semg3, sems0, sems1, sems2, sems3):
    cid = lax.axis_index("c")
    sid = lax.axis_index("s")
    w = cid * NS + sid
    he_b, er_b, p_b = (he0, he1, he2, he3), (er0, er1, er2, er3), \
        (p0, p1, p2, p3)
    semg, sems = (semg0, semg1, semg2, semg3), (sems0, sems1, sems2, sems3)

    zbase = jnp.minimum(sid * DUMP, N - DUMP)
    pltpu.sync_copy(z_hbm.at[pl.ds(zbase, DUMP)], acc.at[pl.ds(zbase, DUMP)])
    pltpu.sync_copy(cvec_hbm, cvec_v)
    pltpu.sync_copy(src_hbm.at[w], srci)
    pltpu.sync_copy(dst_hbm.at[w], dsti)
    plsc.subcore_barrier()

    def issue_g(b, k):
        pltpu.async_copy(he_hbm.at[srci.at[k]], he_b[b], semg[b])
        pltpu.async_copy(er_hbm.at[dsti.at[k]], er_b[b], semg[b])

    def wait_g(b, k):
        pltpu.make_async_copy(he_hbm.at[srci.at[k]], he_b[b], semg[b]).wait()
        pltpu.make_async_copy(er_hbm.at[dsti.at[k]], er_b[b], semg[b]).wait()

    def issue_s(b, k):
        pltpu.async_copy(p_b[b], acc.at[dsti.at[k]], sems[b], add=True)

    def wait_s(b, k):
        pltpu.make_async_copy(
            p_b[b], acc.at[dsti.at[k]], sems[b]).wait()

    def compute(b, k):
        cv = cvec_v[...]
        he_v, er_v, p_v = he_b[b], er_b[b], p_b[b]
        for i in range(CH):
            el = he_v[i, pl.ds(64, 16)]
            er = er_v[i, pl.ds(0, 16)]
            e = el + er
            e = jnp.maximum(e, NEG * e)
            ex = jnp.exp(e - cv)
            p_v[i, pl.ds(0, 16)] = ex
            for q in range(4):
                p_v[i, pl.ds(8 + 16 * q, 16)] = (
                    he_v[i, pl.ds(16 * q, 16)] * ex)

    for kk in range(NBUF - 1):
        issue_g(kk, kk)

    def quad(g, _):
        for b in range(NBUF):
            k = NBUF * g + b
            wait_g(b, k)
            kp = k + NBUF - 1
            pl.when(kp < NCHUNK)(lambda: issue_g((b + NBUF - 1) % NBUF, kp))
            pl.when(g > 0)(lambda: wait_s(b, k))
            compute(b, k)
            issue_s(b, k)
        return ()

    lax.fori_loop(0, NCHUNK // NBUF, quad, (), unroll=False)
    kt = NCHUNK - 1
    for t in range((NCHUNK // NBUF) * NBUF, NCHUNK):
        b = t % NBUF
        wait_g(b, t)
        wait_s(b, t)
        compute(b, t)
        issue_s(b, t)
    for b in range(NBUF):
        wait_s(b, kt)
    plsc.subcore_barrier()
    dbase = jnp.minimum(sid * DUMP, N - DUMP)
    pltpu.sync_copy(acc.at[pl.ds(dbase, DUMP)],
                    out_hbm.at[cid, pl.ds(dbase, DUMP)])


def _sc0_call(he, er, src3, dst3, cvec, z0):
    mesh = plsc.VectorSubcoreMesh(core_axis_name="c", subcore_axis_name="s")
    f = pl.kernel(
        _sc0_body,
        out_type=jax.ShapeDtypeStruct((NC, N, P0W), jnp.float32),
        mesh=mesh,
        compiler_params=_SC_PARAMS,
        scratch_types=[
            pltpu.VMEM_SHARED((N, P0W), jnp.float32),
            pltpu.VMEM((NCHUNK, CH), jnp.int32),
            pltpu.VMEM((NCHUNK, CH), jnp.int32),
        ] + [pltpu.VMEM((CH, HEW), jnp.float32)] * NBUF
        + [pltpu.VMEM((CH, 16), jnp.float32)] * NBUF
        + [pltpu.VMEM((CH, P0W), jnp.float32)] * NBUF
        + [pltpu.VMEM((16,), jnp.float32)]
        + [pltpu.SemaphoreType.DMA] * (2 * NBUF),
    )
    return f(he, er, src3, dst3, cvec, z0)


def _sc1_body(h1_hbm, er1_hbm, src_hbm, dst_hbm, cvec_hbm, z_hbm, out_hbm,
              acc, srci, dsti, h10, h11, h12, h13, er0, er1, er2, er3,
              p0, p1, p2, p3, cvec_v,
              semg0, semg1, semg2, semg3, sems0, sems1, sems2, sems3):
    cid = lax.axis_index("c")
    sid = lax.axis_index("s")
    w = cid * NS + sid
    h1_b, er_b, p_b = (h10, h11, h12, h13), (er0, er1, er2, er3), \
        (p0, p1, p2, p3)
    semg, sems = (semg0, semg1, semg2, semg3), (sems0, sems1, sems2, sems3)

    zbase = jnp.minimum(sid * DUMP, N - DUMP)
    pltpu.sync_copy(z_hbm.at[pl.ds(zbase, DUMP)], acc.at[pl.ds(zbase, DUMP)])
    pltpu.sync_copy(cvec_hbm, cvec_v)
    pltpu.sync_copy(src_hbm.at[w], srci)
    pltpu.sync_copy(dst_hbm.at[w], dsti)
    plsc.subcore_barrier()

    iot = lax.iota(jnp.int32, 16)
    zv = jnp.zeros((16,), jnp.float32)

    def issue_g(b, k):
        pltpu.async_copy(h1_hbm.at[srci.at[k]], h1_b[b], semg[b])
        pltpu.async_copy(er1_hbm.at[dsti.at[k]], er_b[b], semg[b])

    def wait_g(b, k):
        pltpu.make_async_copy(h1_hbm.at[srci.at[k]], h1_b[b], semg[b]).wait()
        pltpu.make_async_copy(er1_hbm.at[dsti.at[k]], er_b[b], semg[b]).wait()

    def issue_s(b, k):
        pltpu.async_copy(p_b[b], acc.at[dsti.at[k]], sems[b], add=True)

    def wait_s(b, k):
        pltpu.make_async_copy(
            p_b[b], acc.at[dsti.at[k]], sems[b]).wait()

    def compute(b, k):
        cv = cvec_v[...]
        cs = cv[0]
        h1_v, er1_v, p_v = h1_b[b], er_b[b], p_b[b]
        for i in range(CH1):
            h3 = h1_v[i, pl.ds(32, 16)]
            erv = er1_v[i, pl.ds(0, 16)]
            e0 = h3[8] + erv[0]
            e0 = jnp.maximum(e0, NEG * e0) - cs
            exv = jnp.exp(jnp.full((16,), e0, jnp.float32))
            p_v[i, pl.ds(0, 16)] = h1_v[i, pl.ds(0, 16)] * exv
            p_v[i, pl.ds(16, 16)] = h1_v[i, pl.ds(16, 16)] * exv
            p_v[i, pl.ds(32, 16)] = jnp.where(
                iot < 8, h3 * exv, jnp.where(iot == 8, exv, zv))

    for kk in range(NBUF - 1):
        issue_g(kk, kk)

    def quad(g, _):
        for b in range(NBUF):
            k = NBUF * g + b
            wait_g(b, k)
            kp = k + NBUF - 1
            pl.when(kp < NCHUNK1)(lambda: issue_g((b + NBUF - 1) % NBUF, kp))
            pl.when(g > 0)(lambda: wait_s(b, k))
            compute(b, k)
            issue_s(b, k)
        return ()

    lax.fori_loop(0, NCHUNK1 // NBUF, quad, (), unroll=False)
    kt = NCHUNK1 - 1
    for t in range((NCHUNK1 // NBUF) * NBUF, NCHUNK1):
        b = t % NBUF
        wait_g(b, t)
        wait_s(b, t)
        compute(b, t)
        issue_s(b, t)
    for b in range(NBUF):
        wait_s(b, kt)
    plsc.subcore_barrier()
    dbase = jnp.minimum(sid * DUMP, N - DUMP)
    pltpu.sync_copy(acc.at[pl.ds(dbase, DUMP)],
                    out_hbm.at[cid, pl.ds(dbase, DUMP)])


def _sc1_call(h1, er1, src3, dst3, cvec, z1):
    mesh = plsc.VectorSubcoreMesh(core_axis_name="c", subcore_axis_name="s")
    f = pl.kernel(
        _sc1_body,
        out_type=jax.ShapeDtypeStruct((NC, N, P1W), jnp.float32),
        mesh=mesh,
        compiler_params=_SC_PARAMS,
        scratch_types=[
            pltpu.VMEM_SHARED((N, P1W), jnp.float32),
            pltpu.VMEM((NCHUNK1, CH1), jnp.int32),
            pltpu.VMEM((NCHUNK1, CH1), jnp.int32),
        ] + [pltpu.VMEM((CH1, P1W), jnp.float32)] * NBUF
        + [pltpu.VMEM((CH1, 16), jnp.float32)] * NBUF
        + [pltpu.VMEM((CH1, P1W), jnp.float32)] * NBUF
        + [pltpu.VMEM((16,), jnp.float32)]
        + [pltpu.SemaphoreType.DMA] * (2 * NBUF),
    )
    return f(h1, er1, src3, dst3, cvec, z1)


def kernel(x, edge_index, W0, attn_l0, attn_r0, b0, W1, attn_l1, attn_r1,
           b1):
    src3 = edge_index[0].reshape(NW, NCHUNK, CH)
    dst3 = edge_index[1].reshape(NW, NCHUNK, CH)
    src3b = edge_index[0].reshape(NW, NCHUNK1, CH1)
    dst3b = edge_index[1].reshape(NW, NCHUNK1, CH1)

    he, er, cvec0 = _k1_call(x, W0, attn_l0, attn_r0)
    z0 = jnp.zeros((N, P0W), jnp.float32)
    acc0 = _sc0_call(he, er, src3, dst3, cvec0.reshape(16), z0)

    h1, er1, cvec1 = _k3_call(acc0, b0, W1, attn_l1, attn_r1)
    z1 = jnp.zeros((N, P1W), jnp.float32)
    acc1 = _sc1_call(h1, er1, src3b, dst3b, cvec1.reshape(16), z1)

    return _k5_call(acc1, b1)
